# Initial kernel scaffold; baseline (speedup 1.0000x reference)
import jax
import jax.numpy as jnp
from jax.experimental import pallas as pl


def kernel(xyz, points, color, colors, params):
    raise NotImplementedError("write your pallas kernel here")


# speedup vs baseline: 5.5496x; 5.5496x over previous
import functools
import jax
import jax.numpy as jnp
from jax import lax
from jax.experimental import pallas as pl
from jax.experimental.pallas import tpu as pltpu
from jax.experimental.pallas import tpu_sc as plsc

_NPOINT = 512
_RADIUS = [0.4, 0.8, 1.6]
_NSAMPLE = [16, 32, 64]
_TABLE_D = 128
_EPS = 1e-5


def _fps_body(xs_ref, ys_ref, zs_ref, out_ref, dist_ref, far_ref):
    B, N = xs_ref.shape
    S = out_ref.shape[1]
    xs = xs_ref[...]
    ys = ys_ref[...]
    zs = zs_ref[...]
    iota = lax.broadcasted_iota(jnp.int32, (B, N), 1)
    iota_s = lax.broadcasted_iota(jnp.int32, (B, S), 1)
    dist_ref[...] = jnp.full((B, N), 1e10, jnp.float32)
    far_ref[...] = jnp.zeros((B, 1), jnp.int32)

    def body(i, carry):
        far = far_ref[...]
        pick = iota == far
        cx = jnp.sum(jnp.where(pick, xs, 0.0), axis=1, keepdims=True)
        cy = jnp.sum(jnp.where(pick, ys, 0.0), axis=1, keepdims=True)
        cz = jnp.sum(jnp.where(pick, zs, 0.0), axis=1, keepdims=True)
        dx = xs - cx
        dy = ys - cy
        dz = zs - cz
        d = (dx * dx + dy * dy) + dz * dz
        dmin = jnp.minimum(dist_ref[...], d)
        dist_ref[...] = dmin
        out_ref[...] = jnp.where(iota_s == i, far, out_ref[...])
        mx = jnp.max(dmin, axis=1, keepdims=True)
        cand = jnp.where(dmin == mx, iota, N)
        far_ref[...] = jnp.min(cand, axis=1, keepdims=True)
        return carry

    lax.fori_loop(0, S, body, 0)


def _fps(xs, ys, zs, npoint):
    B, N = xs.shape
    return pl.pallas_call(
        _fps_body,
        out_shape=jax.ShapeDtypeStruct((B, npoint), jnp.int32),
        scratch_shapes=[
            pltpu.VMEM((B, N), jnp.float32),
            pltpu.VMEM((B, 1), jnp.int32),
        ],
    )(xs, ys, zs)


def _bq_body(xyz_ref, new_ref, gi_ref, sq_ref, d_ref, m_ref, *, r2, K, N):
    x = xyz_ref[0, 0:1, :]
    y = xyz_ref[0, 1:2, :]
    z = xyz_ref[0, 2:3, :]
    nxyz = new_ref[0]
    sx = nxyz[:, 0:1]
    sy = nxyz[:, 1:2]
    sz = nxyz[:, 2:3]
    mm = (sx * x + sy * y) + sz * z
    sn = (sx * sx + sy * sy) + sz * sz
    dn = (x * x + y * y) + z * z
    d = ((-2.0 * mm) + sn) + dn
    d_ref[...] = d
    m_ref[...] = jnp.where(d <= r2, 1.0, 0.0)
    iota = lax.broadcasted_iota(jnp.int32, d.shape, 1)
    iota_k = lax.broadcasted_iota(jnp.int32, (d.shape[0], K), 1)

    def body(k, carry):
        m = m_ref[...]
        cand = jnp.where(m > 0.0, iota, N)
        sel = jnp.min(cand, axis=1, keepdims=True)
        pick = iota == sel
        sqv = jnp.sum(jnp.where(pick, d_ref[...], 0.0), axis=1, keepdims=True)
        gi_ref[0] = jnp.where(iota_k == k, sel, gi_ref[0])
        sq_ref[0] = jnp.where(iota_k == k, sqv, sq_ref[0])
        m_ref[...] = jnp.where(pick, 0.0, m)
        return carry

    lax.fori_loop(0, K, body, 0)

    gi = gi_ref[0]
    sqv = sq_ref[0]
    first = gi[:, 0:1]
    sqf = sqv[:, 0:1]
    empty = gi == N
    gi_ref[0] = jnp.where(empty, jnp.broadcast_to(first, gi.shape), gi)
    sq_ref[0] = jnp.where(empty, jnp.broadcast_to(sqf, sqv.shape), sqv)


def _ball_query(xyz, new_xyz, radius, K):
    B, _, N = xyz.shape
    S = new_xyz.shape[1]
    SB = 128
    body = functools.partial(_bq_body, r2=radius * radius, K=K, N=N)
    return pl.pallas_call(
        body,
        grid=(B, S // SB),
        in_specs=[
            pl.BlockSpec((1, 3, N), lambda b, s: (b, 0, 0)),
            pl.BlockSpec((1, SB, 3), lambda b, s: (b, s, 0)),
        ],
        out_specs=[
            pl.BlockSpec((1, SB, K), lambda b, s: (b, s, 0)),
            pl.BlockSpec((1, SB, K), lambda b, s: (b, s, 0)),
        ],
        out_shape=[
            jax.ShapeDtypeStruct((B, S, K), jnp.int32),
            jax.ShapeDtypeStruct((B, S, K), jnp.float32),
        ],
        scratch_shapes=[
            pltpu.VMEM((SB, N), jnp.float32),
            pltpu.VMEM((SB, N), jnp.float32),
        ],
    )(xyz, new_xyz)


def _gather_rows(table, flat_idx):
    V, D = table.shape
    R = flat_idx.shape[0]
    info = plsc.get_sparse_core_info()
    NC, NS = info.num_cores, info.num_subcores
    NW = NC * NS
    per_w = R // NW
    CSZ = min(512, per_w)
    n_chunks = per_w // CSZ
    mesh = plsc.VectorSubcoreMesh(core_axis_name="c", subcore_axis_name="s")

    @functools.partial(
        pl.kernel,
        mesh=mesh,
        out_type=jax.ShapeDtypeStruct((R, D), jnp.float32),
        scratch_types=[
            pltpu.VMEM((CSZ,), jnp.int32),
            pltpu.VMEM((CSZ, D), jnp.float32),
            pltpu.SemaphoreType.DMA,
        ],
    )
    def gk(table_hbm, idx_hbm, out_hbm, idx_v, rows_v, sem):
        wid = lax.axis_index("s") * NC + lax.axis_index("c")
        base = wid * per_w
        for t in range(n_chunks):
            off = base + t * CSZ
            pltpu.sync_copy(idx_hbm.at[pl.ds(off, CSZ)], idx_v)
            pltpu.async_copy(table_hbm.at[idx_v], rows_v, sem).wait()
            pltpu.sync_copy(rows_v, out_hbm.at[pl.ds(off, CSZ)])

    return gk(table, flat_idx)


def _layer0_body(x_ref, w_ref, b_ref, y_ref, st_ref):
    pi = pl.program_id(0)
    x = x_ref[...]
    y = jnp.dot(w_ref[...], x, preferred_element_type=jnp.float32) + b_ref[...]
    y_ref[...] = y
    s1 = jnp.sum(y, axis=1, keepdims=True)
    s2 = jnp.sum(y * y, axis=1, keepdims=True)
    st = jnp.concatenate([s1, s2], axis=1)

    @pl.when(pi == 0)
    def _():
        st_ref[...] = st

    @pl.when(pi != 0)
    def _():
        st_ref[...] += st


def _layer_act_body(x_ref, w_ref, b_ref, a_ref, c_ref, y_ref, st_ref):
    pi = pl.program_id(0)
    x = jnp.maximum(x_ref[...] * a_ref[...] + c_ref[...], 0.0)
    y = jnp.dot(w_ref[...], x, preferred_element_type=jnp.float32) + b_ref[...]
    y_ref[...] = y
    s1 = jnp.sum(y, axis=1, keepdims=True)
    s2 = jnp.sum(y * y, axis=1, keepdims=True)
    st = jnp.concatenate([s1, s2], axis=1)

    @pl.when(pi == 0)
    def _():
        st_ref[...] = st

    @pl.when(pi != 0)
    def _():
        st_ref[...] += st


def _layer(x, w, b, ac=None, pblk=2048):
    Cin, P = x.shape
    Cout = w.shape[0]
    grid = (P // pblk,)
    outs = [
        jax.ShapeDtypeStruct((Cout, P), jnp.float32),
        jax.ShapeDtypeStruct((Cout, 2), jnp.float32),
    ]
    out_specs = [
        pl.BlockSpec((Cout, pblk), lambda p: (0, p)),
        pl.BlockSpec((Cout, 2), lambda p: (0, 0)),
    ]
    if ac is None:
        return pl.pallas_call(
            _layer0_body,
            grid=grid,
            in_specs=[
                pl.BlockSpec((Cin, pblk), lambda p: (0, p)),
                pl.BlockSpec((Cout, Cin), lambda p: (0, 0)),
                pl.BlockSpec((Cout, 1), lambda p: (0, 0)),
            ],
            out_specs=out_specs,
            out_shape=outs,
        )(x, w, b)
    a, c = ac
    return pl.pallas_call(
        _layer_act_body,
        grid=grid,
        in_specs=[
            pl.BlockSpec((Cin, pblk), lambda p: (0, p)),
            pl.BlockSpec((Cout, Cin), lambda p: (0, 0)),
            pl.BlockSpec((Cout, 1), lambda p: (0, 0)),
            pl.BlockSpec((Cin, 1), lambda p: (0, 0)),
            pl.BlockSpec((Cin, 1), lambda p: (0, 0)),
        ],
        out_specs=out_specs,
        out_shape=outs,
    )(x, w, b, a, c)


def _maxpool_body(y_ref, a_ref, c_ref, o_ref):
    y = y_ref[...]
    a = a_ref[...][:, :, None]
    c = c_ref[...][:, :, None]
    z = jnp.maximum(y * a + c, 0.0)
    o_ref[...] = jnp.max(z, axis=2)


def _maxpool(y, a, c, BS, K, bsb=256):
    C = y.shape[0]
    y3 = y.reshape(C, BS, K)
    return pl.pallas_call(
        _maxpool_body,
        grid=(BS // bsb,),
        in_specs=[
            pl.BlockSpec((C, bsb, K), lambda q: (0, q, 0)),
            pl.BlockSpec((C, 1), lambda q: (0, 0)),
            pl.BlockSpec((C, 1), lambda q: (0, 0)),
        ],
        out_specs=pl.BlockSpec((C, bsb), lambda q: (0, q)),
        out_shape=jax.ShapeDtypeStruct((C, BS), jnp.float32),
    )(y3, a, c)


def _affine_from_stats(st, count):
    mean = st[:, 0:1] / count
    var = st[:, 1:2] / count - mean * mean
    a = lax.rsqrt(var + _EPS)
    return a, -mean * a


def kernel(xyz, points, color, colors, params):
    B, _, N = xyz.shape
    S = _NPOINT
    f32 = jnp.float32

    xyz_t = jnp.transpose(xyz, (0, 2, 1))
    color_t = jnp.transpose(color, (0, 2, 1))
    points_t = jnp.transpose(points, (0, 2, 1))

    fps = _fps(xyz[:, 0, :], xyz[:, 1, :], xyz[:, 2, :], S)
    new_xyz = jnp.take_along_axis(xyz_t, fps[..., None], axis=1)
    new_color = jnp.take_along_axis(color_t, fps[..., None], axis=1)

    pad = jnp.zeros((B, N, _TABLE_D - 3 - points_t.shape[2]), f32)
    table = jnp.concatenate([xyz_t, points_t, pad], axis=2).reshape(B * N, _TABLE_D)
    boff = (jnp.arange(B, dtype=jnp.int32) * N)[:, None, None]

    outs = []
    for i, (radius, K) in enumerate(zip(_RADIUS, _NSAMPLE)):
        gi, sq = _ball_query(xyz, new_xyz, radius, K)
        flat = (gi + boff).reshape(-1)
        rows = _gather_rows(table, flat).reshape(B, S, K, _TABLE_D)
        gxyz = rows[..., :3] - new_xyz[:, :, None, :]
        gpts = rows[..., 3:3 + points_t.shape[2]]
        yuan = jnp.broadcast_to(new_xyz[:, :, None, :], (B, S, K, 3))
        X = jnp.concatenate([gpts, gxyz, yuan, sq[..., None]], axis=-1)
        P = B * S * K
        Xf = jnp.transpose(X, (3, 0, 1, 2)).reshape(X.shape[-1], P)

        ac = None
        y = Xf
        for j, (W, b) in enumerate(params['convs'][i]):
            y, st = _layer(y, W, b[:, None], ac=ac)
            ac = _affine_from_stats(st, float(P))
        pooled = _maxpool(y, ac[0], ac[1], B * S, K)
        outs.append(jnp.transpose(pooled.reshape(-1, B, S), (1, 0, 2)))

    npc = jnp.concatenate(outs, axis=1)
    return (
        jnp.transpose(new_xyz, (0, 2, 1)),
        npc,
        jnp.transpose(new_color, (0, 2, 1)),
        npc,
    )


# speedup vs baseline: 6.2052x; 1.1181x over previous
import functools
import jax
import jax.numpy as jnp
from jax import lax
from jax.experimental import pallas as pl
from jax.experimental.pallas import tpu as pltpu
from jax.experimental.pallas import tpu_sc as plsc

_NPOINT = 512
_RADIUS = [0.4, 0.8, 1.6]
_NSAMPLE = [16, 32, 64]
_TABLE_D = 128
_EPS = 1e-5


def _fps_body(xs_ref, ys_ref, zs_ref, out_ref, dist_ref, far_ref):
    B, R, Cn = xs_ref.shape
    N = R * Cn
    S = out_ref.shape[1]
    xs = xs_ref[...]
    ys = ys_ref[...]
    zs = zs_ref[...]
    n_iota = (lax.broadcasted_iota(jnp.int32, (B, R, Cn), 1) * Cn
              + lax.broadcasted_iota(jnp.int32, (B, R, Cn), 2))
    iota_s = lax.broadcasted_iota(jnp.int32, (B, S), 1)
    dist_ref[...] = jnp.full((B, R, Cn), 1e10, jnp.float32)
    far_ref[...] = jnp.zeros((B, 1, 1), jnp.int32)

    def _red(op, v):
        return op(op(v, axis=2, keepdims=True), axis=1, keepdims=True)

    def body(i, carry):
        far = far_ref[...]
        pick = n_iota == far
        cx = _red(jnp.sum, jnp.where(pick, xs, 0.0))
        cy = _red(jnp.sum, jnp.where(pick, ys, 0.0))
        cz = _red(jnp.sum, jnp.where(pick, zs, 0.0))
        dx = xs - cx
        dy = ys - cy
        dz = zs - cz
        d = (dx * dx + dy * dy) + dz * dz
        dmin = jnp.minimum(dist_ref[...], d)
        dist_ref[...] = dmin
        out_ref[...] = jnp.where(iota_s == i, far[:, 0, :], out_ref[...])
        mx = _red(jnp.max, dmin)
        cand = jnp.where(dmin == mx, n_iota, N)
        far_ref[...] = _red(jnp.min, cand)
        return carry

    lax.fori_loop(0, S, body, 0)


def _fps(xs, ys, zs, npoint):
    B, R, Cn = xs.shape
    return pl.pallas_call(
        _fps_body,
        out_shape=jax.ShapeDtypeStruct((B, npoint), jnp.int32),
        scratch_shapes=[
            pltpu.VMEM((B, R, Cn), jnp.float32),
            pltpu.VMEM((B, 1, 1), jnp.int32),
        ],
    )(xs, ys, zs)


def _bq_body(xyz_ref, new_ref, gi_ref, c_ref, *, r2, K, N):
    x = xyz_ref[0, 0:1, :]
    y = xyz_ref[0, 1:2, :]
    z = xyz_ref[0, 2:3, :]
    nxyz = new_ref[0]
    sx = nxyz[:, 0:1]
    sy = nxyz[:, 1:2]
    sz = nxyz[:, 2:3]
    mm = jnp.dot(nxyz, xyz_ref[0], preferred_element_type=jnp.float32)
    sn = (sx * sx + sy * sy) + sz * sz
    dn = (x * x + y * y) + z * z
    d = ((-2.0 * mm) + sn) + dn
    iota = lax.broadcasted_iota(jnp.int32, d.shape, 1)
    c_ref[...] = jnp.where(d <= r2, iota, N)
    iota_k = lax.broadcasted_iota(jnp.int32, (d.shape[0], K), 1)

    def body(k, carry):
        c = c_ref[...]
        sel = jnp.min(c, axis=1, keepdims=True)
        gi_ref[0] = jnp.where(iota_k == k, sel, gi_ref[0])
        c_ref[...] = jnp.where(c == sel, N, c)
        return carry

    lax.fori_loop(0, K, body, 0)

    gi = gi_ref[0]
    first = gi[:, 0:1]
    gi_ref[0] = jnp.where(gi == N, jnp.broadcast_to(first, gi.shape), gi)


def _ball_query(xyz, new_xyz, radius, K):
    B, _, N = xyz.shape
    S = new_xyz.shape[1]
    SB = 128
    body = functools.partial(_bq_body, r2=radius * radius, K=K, N=N)
    return pl.pallas_call(
        body,
        grid=(B, S // SB),
        in_specs=[
            pl.BlockSpec((1, 3, N), lambda b, s: (b, 0, 0)),
            pl.BlockSpec((1, SB, 3), lambda b, s: (b, s, 0)),
        ],
        out_specs=pl.BlockSpec((1, SB, K), lambda b, s: (b, s, 0)),
        out_shape=jax.ShapeDtypeStruct((B, S, K), jnp.int32),
        scratch_shapes=[
            pltpu.VMEM((SB, N), jnp.int32),
        ],
    )(xyz, new_xyz)


def _gather_rows(table, flat_idx):
    V, D = table.shape
    R = flat_idx.shape[0]
    info = plsc.get_sparse_core_info()
    NC, NS = info.num_cores, info.num_subcores
    NW = NC * NS
    per_w = R // NW
    CSZ = min(512, per_w)
    n_chunks = per_w // CSZ
    mesh = plsc.VectorSubcoreMesh(core_axis_name="c", subcore_axis_name="s")

    @functools.partial(
        pl.kernel,
        mesh=mesh,
        out_type=jax.ShapeDtypeStruct((R, D), jnp.float32),
        scratch_types=[
            pltpu.VMEM((CSZ,), jnp.int32),
            pltpu.VMEM((CSZ, D), jnp.float32),
            pltpu.SemaphoreType.DMA,
        ],
    )
    def gk(table_hbm, idx_hbm, out_hbm, idx_v, rows_v, sem):
        wid = lax.axis_index("s") * NC + lax.axis_index("c")
        base = wid * per_w
        for t in range(n_chunks):
            off = base + t * CSZ
            pltpu.sync_copy(idx_hbm.at[pl.ds(off, CSZ)], idx_v)
            pltpu.async_copy(table_hbm.at[idx_v], rows_v, sem).wait()
            pltpu.sync_copy(rows_v, out_hbm.at[pl.ds(off, CSZ)])

    return gk(table, flat_idx)


def _layer0_body(x_ref, w_ref, b_ref, y_ref, st_ref):
    pi = pl.program_id(0)
    x = x_ref[...]
    y = jnp.dot(w_ref[...], x, preferred_element_type=jnp.float32) + b_ref[...]
    y_ref[...] = y
    s1 = jnp.sum(y, axis=1, keepdims=True)
    s2 = jnp.sum(y * y, axis=1, keepdims=True)
    st = jnp.concatenate([s1, s2], axis=1)

    @pl.when(pi == 0)
    def _():
        st_ref[...] = st

    @pl.when(pi != 0)
    def _():
        st_ref[...] += st


def _layer_act_body(x_ref, w_ref, b_ref, a_ref, c_ref, y_ref, st_ref):
    pi = pl.program_id(0)
    x = jnp.maximum(x_ref[...] * a_ref[...] + c_ref[...], 0.0)
    y = jnp.dot(w_ref[...], x, preferred_element_type=jnp.float32) + b_ref[...]
    y_ref[...] = y
    s1 = jnp.sum(y, axis=1, keepdims=True)
    s2 = jnp.sum(y * y, axis=1, keepdims=True)
    st = jnp.concatenate([s1, s2], axis=1)

    @pl.when(pi == 0)
    def _():
        st_ref[...] = st

    @pl.when(pi != 0)
    def _():
        st_ref[...] += st


def _layer(x, w, b, ac=None, pblk=2048):
    Cin, P = x.shape
    Cout = w.shape[0]
    grid = (P // pblk,)
    outs = [
        jax.ShapeDtypeStruct((Cout, P), jnp.float32),
        jax.ShapeDtypeStruct((Cout, 2), jnp.float32),
    ]
    out_specs = [
        pl.BlockSpec((Cout, pblk), lambda p: (0, p)),
        pl.BlockSpec((Cout, 2), lambda p: (0, 0)),
    ]
    if ac is None:
        return pl.pallas_call(
            _layer0_body,
            grid=grid,
            in_specs=[
                pl.BlockSpec((Cin, pblk), lambda p: (0, p)),
                pl.BlockSpec((Cout, Cin), lambda p: (0, 0)),
                pl.BlockSpec((Cout, 1), lambda p: (0, 0)),
            ],
            out_specs=out_specs,
            out_shape=outs,
        )(x, w, b)
    a, c = ac
    return pl.pallas_call(
        _layer_act_body,
        grid=grid,
        in_specs=[
            pl.BlockSpec((Cin, pblk), lambda p: (0, p)),
            pl.BlockSpec((Cout, Cin), lambda p: (0, 0)),
            pl.BlockSpec((Cout, 1), lambda p: (0, 0)),
            pl.BlockSpec((Cin, 1), lambda p: (0, 0)),
            pl.BlockSpec((Cin, 1), lambda p: (0, 0)),
        ],
        out_specs=out_specs,
        out_shape=outs,
    )(x, w, b, a, c)


def _maxpool_body(y_ref, a_ref, c_ref, o_ref):
    y = y_ref[...]
    a = a_ref[...][:, :, None]
    c = c_ref[...][:, :, None]
    z = jnp.maximum(y * a + c, 0.0)
    o_ref[...] = jnp.max(z, axis=2)


def _maxpool(y, a, c, BS, K, bsb=256):
    C = y.shape[0]
    y3 = y.reshape(C, BS, K)
    return pl.pallas_call(
        _maxpool_body,
        grid=(BS // bsb,),
        in_specs=[
            pl.BlockSpec((C, bsb, K), lambda q: (0, q, 0)),
            pl.BlockSpec((C, 1), lambda q: (0, 0)),
            pl.BlockSpec((C, 1), lambda q: (0, 0)),
        ],
        out_specs=pl.BlockSpec((C, bsb), lambda q: (0, q)),
        out_shape=jax.ShapeDtypeStruct((C, BS), jnp.float32),
    )(y3, a, c)


def _affine_from_stats(st, count):
    mean = st[:, 0:1] / count
    var = st[:, 1:2] / count - mean * mean
    a = lax.rsqrt(var + _EPS)
    return a, -mean * a


def kernel(xyz, points, color, colors, params):
    B, _, N = xyz.shape
    S = _NPOINT
    f32 = jnp.float32

    xyz_t = jnp.transpose(xyz, (0, 2, 1))
    color_t = jnp.transpose(color, (0, 2, 1))
    points_t = jnp.transpose(points, (0, 2, 1))

    fps = _fps(xyz[:, 0, :].reshape(B, 8, N // 8),
               xyz[:, 1, :].reshape(B, 8, N // 8),
               xyz[:, 2, :].reshape(B, 8, N // 8), S)
    new_xyz = jnp.take_along_axis(xyz_t, fps[..., None], axis=1)
    new_color = jnp.take_along_axis(color_t, fps[..., None], axis=1)

    px, py, pz = xyz_t[..., 0], xyz_t[..., 1], xyz_t[..., 2]
    norm2 = ((px * px + py * py) + pz * pz)[..., None]
    pad = jnp.zeros((B, N, _TABLE_D - 4 - points_t.shape[2]), f32)
    table = jnp.concatenate([xyz_t, norm2, points_t, pad], axis=2).reshape(B * N, _TABLE_D)
    boff = (jnp.arange(B, dtype=jnp.int32) * N)[:, None, None]
    sx, sy, sz = new_xyz[..., 0:1], new_xyz[..., 1:2], new_xyz[..., 2:3]
    sn = (sx * sx + sy * sy) + sz * sz

    outs = []
    for i, (radius, K) in enumerate(zip(_RADIUS, _NSAMPLE)):
        gi = _ball_query(xyz, new_xyz, radius, K)
        flat = (gi + boff).reshape(-1)
        rows = _gather_rows(table, flat).reshape(B, S, K, _TABLE_D)
        gx, gy, gz = rows[..., 0], rows[..., 1], rows[..., 2]
        mm = (sx * gx + sy * gy) + sz * gz
        sq = ((-2.0 * mm) + sn) + rows[..., 3]
        gxyz = rows[..., :3] - new_xyz[:, :, None, :]
        gpts = rows[..., 4:4 + points_t.shape[2]]
        yuan = jnp.broadcast_to(new_xyz[:, :, None, :], (B, S, K, 3))
        X = jnp.concatenate([gpts, gxyz, yuan, sq[..., None]], axis=-1)
        P = B * S * K
        Xf = jnp.transpose(X, (3, 0, 1, 2)).reshape(X.shape[-1], P)

        ac = None
        y = Xf
        for j, (W, b) in enumerate(params['convs'][i]):
            y, st = _layer(y, W, b[:, None], ac=ac)
            ac = _affine_from_stats(st, float(P))
        pooled = _maxpool(y, ac[0], ac[1], B * S, K)
        outs.append(jnp.transpose(pooled.reshape(-1, B, S), (1, 0, 2)))

    npc = jnp.concatenate(outs, axis=1)
    return (
        jnp.transpose(new_xyz, (0, 2, 1)),
        npc,
        jnp.transpose(new_color, (0, 2, 1)),
        npc,
    )


# speedup vs baseline: 6.4428x; 1.0383x over previous
import functools
import jax
import jax.numpy as jnp
from jax import lax
from jax.experimental import pallas as pl
from jax.experimental.pallas import tpu as pltpu
from jax.experimental.pallas import tpu_sc as plsc

_NPOINT = 512
_RADIUS = [0.4, 0.8, 1.6]
_NSAMPLE = [16, 32, 64]
_TABLE_D = 128
_EPS = 1e-5


def _fps_body(xs_ref, ys_ref, zs_ref, out_ref, dist_ref, far_ref):
    B, R, Cn = xs_ref.shape
    N = R * Cn
    S = out_ref.shape[1]
    xs = xs_ref[...]
    ys = ys_ref[...]
    zs = zs_ref[...]
    n_iota = (lax.broadcasted_iota(jnp.int32, (B, R, Cn), 1) * Cn
              + lax.broadcasted_iota(jnp.int32, (B, R, Cn), 2))
    iota_s = lax.broadcasted_iota(jnp.int32, (B, S), 1)
    dist_ref[...] = jnp.full((B, R, Cn), 1e10, jnp.float32)
    far_ref[...] = jnp.zeros((B, 1, 1), jnp.int32)

    def _red(op, v):
        return op(op(v, axis=2, keepdims=True), axis=1, keepdims=True)

    def body(i, carry):
        far = far_ref[...]
        pick = n_iota == far
        cx = _red(jnp.sum, jnp.where(pick, xs, 0.0))
        cy = _red(jnp.sum, jnp.where(pick, ys, 0.0))
        cz = _red(jnp.sum, jnp.where(pick, zs, 0.0))
        dx = xs - cx
        dy = ys - cy
        dz = zs - cz
        d = (dx * dx + dy * dy) + dz * dz
        dmin = jnp.minimum(dist_ref[...], d)
        dist_ref[...] = dmin
        out_ref[...] = jnp.where(iota_s == i, far[:, 0, :], out_ref[...])
        mx = _red(jnp.max, dmin)
        cand = jnp.where(dmin == mx, n_iota, N)
        far_ref[...] = _red(jnp.min, cand)
        return carry

    lax.fori_loop(0, S, body, 0, unroll=2)


def _fps(xs, ys, zs, npoint):
    B, R, Cn = xs.shape
    return pl.pallas_call(
        _fps_body,
        out_shape=jax.ShapeDtypeStruct((B, npoint), jnp.int32),
        scratch_shapes=[
            pltpu.VMEM((B, R, Cn), jnp.float32),
            pltpu.VMEM((B, 1, 1), jnp.int32),
        ],
    )(xs, ys, zs)


def _bq_body(xyz_ref, new_ref, gi_ref, c_ref, *, r2, K, N):
    x = xyz_ref[0, 0:1, :]
    y = xyz_ref[0, 1:2, :]
    z = xyz_ref[0, 2:3, :]
    nxyz = new_ref[0]
    sx = nxyz[:, 0:1]
    sy = nxyz[:, 1:2]
    sz = nxyz[:, 2:3]
    mm = jnp.dot(nxyz, xyz_ref[0], preferred_element_type=jnp.float32)
    sn = (sx * sx + sy * sy) + sz * sz
    dn = (x * x + y * y) + z * z
    d = ((-2.0 * mm) + sn) + dn
    iota = lax.broadcasted_iota(jnp.int32, d.shape, 1)
    c_ref[...] = jnp.where(d <= r2, iota, N)
    iota_k = lax.broadcasted_iota(jnp.int32, (d.shape[0], K), 1)

    def body(k, carry):
        c = c_ref[...]
        sel = jnp.min(c, axis=1, keepdims=True)
        gi_ref[0] = jnp.where(iota_k == k, sel, gi_ref[0])
        c_ref[...] = jnp.where(c == sel, N, c)
        return carry

    lax.fori_loop(0, K, body, 0, unroll=2)

    gi = gi_ref[0]
    first = gi[:, 0:1]
    gi_ref[0] = jnp.where(gi == N, jnp.broadcast_to(first, gi.shape), gi)


def _ball_query(xyz, new_xyz, radius, K):
    B, _, N = xyz.shape
    S = new_xyz.shape[1]
    SB = 256
    body = functools.partial(_bq_body, r2=radius * radius, K=K, N=N)
    return pl.pallas_call(
        body,
        grid=(B, S // SB),
        in_specs=[
            pl.BlockSpec((1, 3, N), lambda b, s: (b, 0, 0)),
            pl.BlockSpec((1, SB, 3), lambda b, s: (b, s, 0)),
        ],
        out_specs=pl.BlockSpec((1, SB, K), lambda b, s: (b, s, 0)),
        out_shape=jax.ShapeDtypeStruct((B, S, K), jnp.int32),
        scratch_shapes=[
            pltpu.VMEM((SB, N), jnp.int32),
        ],
    )(xyz, new_xyz)


def _gather_rows(table, flat_idx):
    V, D = table.shape
    R = flat_idx.shape[0]
    info = plsc.get_sparse_core_info()
    NC, NS = info.num_cores, info.num_subcores
    NW = NC * NS
    per_w = R // NW
    CSZ = min(512, per_w)
    n_chunks = per_w // CSZ
    mesh = plsc.VectorSubcoreMesh(core_axis_name="c", subcore_axis_name="s")

    @functools.partial(
        pl.kernel,
        mesh=mesh,
        out_type=jax.ShapeDtypeStruct((R, D), jnp.float32),
        scratch_types=[
            pltpu.VMEM((CSZ,), jnp.int32),
            pltpu.VMEM((CSZ, D), jnp.float32),
            pltpu.SemaphoreType.DMA,
        ],
    )
    def gk(table_hbm, idx_hbm, out_hbm, idx_v, rows_v, sem):
        wid = lax.axis_index("s") * NC + lax.axis_index("c")
        base = wid * per_w
        for t in range(n_chunks):
            off = base + t * CSZ
            pltpu.sync_copy(idx_hbm.at[pl.ds(off, CSZ)], idx_v)
            pltpu.async_copy(table_hbm.at[idx_v], rows_v, sem).wait()
            pltpu.sync_copy(rows_v, out_hbm.at[pl.ds(off, CSZ)])

    return gk(table, flat_idx)


def _layer0_body(x_ref, w_ref, b_ref, y_ref, st_ref):
    pi = pl.program_id(0)
    x = x_ref[...]
    y = jnp.dot(w_ref[...], x, preferred_element_type=jnp.float32) + b_ref[...]
    y_ref[...] = y
    s1 = jnp.sum(y, axis=1, keepdims=True)
    s2 = jnp.sum(y * y, axis=1, keepdims=True)
    st = jnp.concatenate([s1, s2], axis=1)

    @pl.when(pi == 0)
    def _():
        st_ref[...] = st

    @pl.when(pi != 0)
    def _():
        st_ref[...] += st


def _layer_act_body(x_ref, w_ref, b_ref, a_ref, c_ref, y_ref, st_ref):
    pi = pl.program_id(0)
    x = jnp.maximum(x_ref[...] * a_ref[...] + c_ref[...], 0.0)
    y = jnp.dot(w_ref[...], x, preferred_element_type=jnp.float32) + b_ref[...]
    y_ref[...] = y
    s1 = jnp.sum(y, axis=1, keepdims=True)
    s2 = jnp.sum(y * y, axis=1, keepdims=True)
    st = jnp.concatenate([s1, s2], axis=1)

    @pl.when(pi == 0)
    def _():
        st_ref[...] = st

    @pl.when(pi != 0)
    def _():
        st_ref[...] += st


def _layer(x, w, b, ac=None, pblk=2048):
    Cin, P = x.shape
    Cout = w.shape[0]
    grid = (P // pblk,)
    outs = [
        jax.ShapeDtypeStruct((Cout, P), jnp.float32),
        jax.ShapeDtypeStruct((Cout, 2), jnp.float32),
    ]
    out_specs = [
        pl.BlockSpec((Cout, pblk), lambda p: (0, p)),
        pl.BlockSpec((Cout, 2), lambda p: (0, 0)),
    ]
    if ac is None:
        return pl.pallas_call(
            _layer0_body,
            grid=grid,
            in_specs=[
                pl.BlockSpec((Cin, pblk), lambda p: (0, p)),
                pl.BlockSpec((Cout, Cin), lambda p: (0, 0)),
                pl.BlockSpec((Cout, 1), lambda p: (0, 0)),
            ],
            out_specs=out_specs,
            out_shape=outs,
        )(x, w, b)
    a, c = ac
    return pl.pallas_call(
        _layer_act_body,
        grid=grid,
        in_specs=[
            pl.BlockSpec((Cin, pblk), lambda p: (0, p)),
            pl.BlockSpec((Cout, Cin), lambda p: (0, 0)),
            pl.BlockSpec((Cout, 1), lambda p: (0, 0)),
            pl.BlockSpec((Cin, 1), lambda p: (0, 0)),
            pl.BlockSpec((Cin, 1), lambda p: (0, 0)),
        ],
        out_specs=out_specs,
        out_shape=outs,
    )(x, w, b, a, c)


def _maxpool_body(y_ref, a_ref, c_ref, o_ref):
    y = y_ref[...]
    a = a_ref[...][:, :, None]
    c = c_ref[...][:, :, None]
    z = jnp.maximum(y * a + c, 0.0)
    o_ref[...] = jnp.max(z, axis=2)


def _maxpool(y, a, c, BS, K, bsb=256):
    C = y.shape[0]
    y3 = y.reshape(C, BS, K)
    return pl.pallas_call(
        _maxpool_body,
        grid=(BS // bsb,),
        in_specs=[
            pl.BlockSpec((C, bsb, K), lambda q: (0, q, 0)),
            pl.BlockSpec((C, 1), lambda q: (0, 0)),
            pl.BlockSpec((C, 1), lambda q: (0, 0)),
        ],
        out_specs=pl.BlockSpec((C, bsb), lambda q: (0, q)),
        out_shape=jax.ShapeDtypeStruct((C, BS), jnp.float32),
    )(y3, a, c)


def _affine_from_stats(st, count):
    mean = st[:, 0:1] / count
    var = st[:, 1:2] / count - mean * mean
    a = lax.rsqrt(var + _EPS)
    return a, -mean * a


def kernel(xyz, points, color, colors, params):
    B, _, N = xyz.shape
    S = _NPOINT
    f32 = jnp.float32

    xyz_t = jnp.transpose(xyz, (0, 2, 1))
    color_t = jnp.transpose(color, (0, 2, 1))
    points_t = jnp.transpose(points, (0, 2, 1))

    fps = _fps(xyz[:, 0, :].reshape(B, 8, N // 8),
               xyz[:, 1, :].reshape(B, 8, N // 8),
               xyz[:, 2, :].reshape(B, 8, N // 8), S)
    new_xyz = jnp.take_along_axis(xyz_t, fps[..., None], axis=1)
    new_color = jnp.take_along_axis(color_t, fps[..., None], axis=1)

    px, py, pz = xyz_t[..., 0], xyz_t[..., 1], xyz_t[..., 2]
    norm2 = ((px * px + py * py) + pz * pz)[..., None]
    pad = jnp.zeros((B, N, _TABLE_D - 4 - points_t.shape[2]), f32)
    table = jnp.concatenate([xyz_t, norm2, points_t, pad], axis=2).reshape(B * N, _TABLE_D)
    boff = (jnp.arange(B, dtype=jnp.int32) * N)[:, None, None]
    sx, sy, sz = new_xyz[..., 0:1], new_xyz[..., 1:2], new_xyz[..., 2:3]
    sn = (sx * sx + sy * sy) + sz * sz

    outs = []
    for i, (radius, K) in enumerate(zip(_RADIUS, _NSAMPLE)):
        gi = _ball_query(xyz, new_xyz, radius, K)
        flat = (gi + boff).reshape(-1)
        rows = _gather_rows(table, flat).reshape(B, S, K, _TABLE_D)
        gx, gy, gz = rows[..., 0], rows[..., 1], rows[..., 2]
        mm = (sx * gx + sy * gy) + sz * gz
        sq = ((-2.0 * mm) + sn) + rows[..., 3]
        gxyz = rows[..., :3] - new_xyz[:, :, None, :]
        gpts = rows[..., 4:4 + points_t.shape[2]]
        yuan = jnp.broadcast_to(new_xyz[:, :, None, :], (B, S, K, 3))
        X = jnp.concatenate([gpts, gxyz, yuan, sq[..., None]], axis=-1)
        P = B * S * K
        Xf = jnp.transpose(X, (3, 0, 1, 2)).reshape(X.shape[-1], P)

        ac = None
        y = Xf
        for j, (W, b) in enumerate(params['convs'][i]):
            y, st = _layer(y, W, b[:, None], ac=ac)
            ac = _affine_from_stats(st, float(P))
        pooled = _maxpool(y, ac[0], ac[1], B * S, K)
        outs.append(jnp.transpose(pooled.reshape(-1, B, S), (1, 0, 2)))

    npc = jnp.concatenate(outs, axis=1)
    return (
        jnp.transpose(new_xyz, (0, 2, 1)),
        npc,
        jnp.transpose(new_color, (0, 2, 1)),
        npc,
    )


# speedup vs baseline: 6.4567x; 1.0021x over previous
import functools
import jax
import jax.numpy as jnp
from jax import lax
from jax.experimental import pallas as pl
from jax.experimental.pallas import tpu as pltpu
from jax.experimental.pallas import tpu_sc as plsc

_NPOINT = 512
_RADIUS = [0.4, 0.8, 1.6]
_NSAMPLE = [16, 32, 64]
_TABLE_D = 128
_EPS = 1e-5


def _fps_body(xs_ref, ys_ref, zs_ref, out_ref, dist_ref, far_ref):
    B, R, Cn = xs_ref.shape
    N = R * Cn
    S = out_ref.shape[1]
    xs = xs_ref[...]
    ys = ys_ref[...]
    zs = zs_ref[...]
    n_iota = (lax.broadcasted_iota(jnp.int32, (B, R, Cn), 1) * Cn
              + lax.broadcasted_iota(jnp.int32, (B, R, Cn), 2))
    iota_s = lax.broadcasted_iota(jnp.int32, (B, S), 1)
    dist_ref[...] = jnp.full((B, R, Cn), 1e10, jnp.float32)
    far_ref[...] = jnp.zeros((B, 1, 1), jnp.int32)

    def _red(op, v):
        return op(op(v, axis=2, keepdims=True), axis=1, keepdims=True)

    def body(i, carry):
        far = far_ref[...]
        pick = n_iota == far
        cx = _red(jnp.sum, jnp.where(pick, xs, 0.0))
        cy = _red(jnp.sum, jnp.where(pick, ys, 0.0))
        cz = _red(jnp.sum, jnp.where(pick, zs, 0.0))
        dx = xs - cx
        dy = ys - cy
        dz = zs - cz
        d = (dx * dx + dy * dy) + dz * dz
        dmin = jnp.minimum(dist_ref[...], d)
        dist_ref[...] = dmin
        out_ref[...] = jnp.where(iota_s == i, far[:, 0, :], out_ref[...])
        mx = _red(jnp.max, dmin)
        cand = jnp.where(dmin == mx, n_iota, N)
        far_ref[...] = _red(jnp.min, cand)
        return carry

    lax.fori_loop(0, S, body, 0, unroll=4)


def _fps(xs, ys, zs, npoint):
    B, R, Cn = xs.shape
    return pl.pallas_call(
        _fps_body,
        out_shape=jax.ShapeDtypeStruct((B, npoint), jnp.int32),
        scratch_shapes=[
            pltpu.VMEM((B, R, Cn), jnp.float32),
            pltpu.VMEM((B, 1, 1), jnp.int32),
        ],
    )(xs, ys, zs)


def _bq_body(xyz_ref, new_ref, gi_ref, c_ref, *, r2, K, N):
    x = xyz_ref[0, 0:1, :]
    y = xyz_ref[0, 1:2, :]
    z = xyz_ref[0, 2:3, :]
    nxyz = new_ref[0]
    sx = nxyz[:, 0:1]
    sy = nxyz[:, 1:2]
    sz = nxyz[:, 2:3]
    mm = jnp.dot(nxyz, xyz_ref[0], preferred_element_type=jnp.float32)
    sn = (sx * sx + sy * sy) + sz * sz
    dn = (x * x + y * y) + z * z
    d = ((-2.0 * mm) + sn) + dn
    iota = lax.broadcasted_iota(jnp.int32, d.shape, 1)
    c_ref[...] = jnp.where(d <= r2, iota, N)
    iota_k = lax.broadcasted_iota(jnp.int32, (d.shape[0], K), 1)

    def body(k, carry):
        c = c_ref[...]
        sel = jnp.min(c, axis=1, keepdims=True)
        gi_ref[0] = jnp.where(iota_k == k, sel, gi_ref[0])
        c_ref[...] = jnp.where(c == sel, N, c)
        return carry

    lax.fori_loop(0, K, body, 0, unroll=2)

    gi = gi_ref[0]
    first = gi[:, 0:1]
    gi_ref[0] = jnp.where(gi == N, jnp.broadcast_to(first, gi.shape), gi)


def _ball_query(xyz, new_xyz, radius, K):
    B, _, N = xyz.shape
    S = new_xyz.shape[1]
    SB = 256
    body = functools.partial(_bq_body, r2=radius * radius, K=K, N=N)
    return pl.pallas_call(
        body,
        grid=(B, S // SB),
        in_specs=[
            pl.BlockSpec((1, 3, N), lambda b, s: (b, 0, 0)),
            pl.BlockSpec((1, SB, 3), lambda b, s: (b, s, 0)),
        ],
        out_specs=pl.BlockSpec((1, SB, K), lambda b, s: (b, s, 0)),
        out_shape=jax.ShapeDtypeStruct((B, S, K), jnp.int32),
        scratch_shapes=[
            pltpu.VMEM((SB, N), jnp.int32),
        ],
    )(xyz, new_xyz)


def _gather_rows(table, flat_idx):
    V, D = table.shape
    R = flat_idx.shape[0]
    info = plsc.get_sparse_core_info()
    NC, NS = info.num_cores, info.num_subcores
    NW = NC * NS
    per_w = R // NW
    CSZ = min(512, per_w)
    n_chunks = per_w // CSZ
    mesh = plsc.VectorSubcoreMesh(core_axis_name="c", subcore_axis_name="s")

    @functools.partial(
        pl.kernel,
        mesh=mesh,
        out_type=jax.ShapeDtypeStruct((R, D), jnp.float32),
        scratch_types=[
            pltpu.VMEM((CSZ,), jnp.int32),
            pltpu.VMEM((CSZ, D), jnp.float32),
            pltpu.SemaphoreType.DMA,
        ],
    )
    def gk(table_hbm, idx_hbm, out_hbm, idx_v, rows_v, sem):
        wid = lax.axis_index("s") * NC + lax.axis_index("c")
        base = wid * per_w
        for t in range(n_chunks):
            off = base + t * CSZ
            pltpu.sync_copy(idx_hbm.at[pl.ds(off, CSZ)], idx_v)
            pltpu.async_copy(table_hbm.at[idx_v], rows_v, sem).wait()
            pltpu.sync_copy(rows_v, out_hbm.at[pl.ds(off, CSZ)])

    return gk(table, flat_idx)


def _layer0_body(x_ref, w_ref, b_ref, y_ref, st_ref):
    pi = pl.program_id(0)
    x = x_ref[...]
    y = jnp.dot(w_ref[...], x, preferred_element_type=jnp.float32) + b_ref[...]
    y_ref[...] = y
    s1 = jnp.sum(y, axis=1, keepdims=True)
    s2 = jnp.sum(y * y, axis=1, keepdims=True)
    st = jnp.concatenate([s1, s2], axis=1)

    @pl.when(pi == 0)
    def _():
        st_ref[...] = st

    @pl.when(pi != 0)
    def _():
        st_ref[...] += st


def _layer_act_body(x_ref, w_ref, b_ref, a_ref, c_ref, y_ref, st_ref):
    pi = pl.program_id(0)
    x = jnp.maximum(x_ref[...] * a_ref[...] + c_ref[...], 0.0)
    y = jnp.dot(w_ref[...], x, preferred_element_type=jnp.float32) + b_ref[...]
    y_ref[...] = y
    s1 = jnp.sum(y, axis=1, keepdims=True)
    s2 = jnp.sum(y * y, axis=1, keepdims=True)
    st = jnp.concatenate([s1, s2], axis=1)

    @pl.when(pi == 0)
    def _():
        st_ref[...] = st

    @pl.when(pi != 0)
    def _():
        st_ref[...] += st


def _layer(x, w, b, ac=None, pblk=2048):
    Cin, P = x.shape
    Cout = w.shape[0]
    grid = (P // pblk,)
    outs = [
        jax.ShapeDtypeStruct((Cout, P), jnp.float32),
        jax.ShapeDtypeStruct((Cout, 2), jnp.float32),
    ]
    out_specs = [
        pl.BlockSpec((Cout, pblk), lambda p: (0, p)),
        pl.BlockSpec((Cout, 2), lambda p: (0, 0)),
    ]
    if ac is None:
        return pl.pallas_call(
            _layer0_body,
            grid=grid,
            in_specs=[
                pl.BlockSpec((Cin, pblk), lambda p: (0, p)),
                pl.BlockSpec((Cout, Cin), lambda p: (0, 0)),
                pl.BlockSpec((Cout, 1), lambda p: (0, 0)),
            ],
            out_specs=out_specs,
            out_shape=outs,
        )(x, w, b)
    a, c = ac
    return pl.pallas_call(
        _layer_act_body,
        grid=grid,
        in_specs=[
            pl.BlockSpec((Cin, pblk), lambda p: (0, p)),
            pl.BlockSpec((Cout, Cin), lambda p: (0, 0)),
            pl.BlockSpec((Cout, 1), lambda p: (0, 0)),
            pl.BlockSpec((Cin, 1), lambda p: (0, 0)),
            pl.BlockSpec((Cin, 1), lambda p: (0, 0)),
        ],
        out_specs=out_specs,
        out_shape=outs,
    )(x, w, b, a, c)


def _maxpool_body(y_ref, a_ref, c_ref, o_ref):
    y = y_ref[...]
    a = a_ref[...][:, :, None]
    c = c_ref[...][:, :, None]
    z = jnp.maximum(y * a + c, 0.0)
    o_ref[...] = jnp.max(z, axis=2)


def _maxpool(y, a, c, BS, K, bsb=256):
    C = y.shape[0]
    y3 = y.reshape(C, BS, K)
    return pl.pallas_call(
        _maxpool_body,
        grid=(BS // bsb,),
        in_specs=[
            pl.BlockSpec((C, bsb, K), lambda q: (0, q, 0)),
            pl.BlockSpec((C, 1), lambda q: (0, 0)),
            pl.BlockSpec((C, 1), lambda q: (0, 0)),
        ],
        out_specs=pl.BlockSpec((C, bsb), lambda q: (0, q)),
        out_shape=jax.ShapeDtypeStruct((C, BS), jnp.float32),
    )(y3, a, c)


def _affine_from_stats(st, count):
    mean = st[:, 0:1] / count
    var = st[:, 1:2] / count - mean * mean
    a = lax.rsqrt(var + _EPS)
    return a, -mean * a


def kernel(xyz, points, color, colors, params):
    B, _, N = xyz.shape
    S = _NPOINT
    f32 = jnp.float32

    xyz_t = jnp.transpose(xyz, (0, 2, 1))
    color_t = jnp.transpose(color, (0, 2, 1))
    points_t = jnp.transpose(points, (0, 2, 1))

    fps = _fps(xyz[:, 0, :].reshape(B, 8, N // 8),
               xyz[:, 1, :].reshape(B, 8, N // 8),
               xyz[:, 2, :].reshape(B, 8, N // 8), S)
    new_xyz = jnp.take_along_axis(xyz_t, fps[..., None], axis=1)
    new_color = jnp.take_along_axis(color_t, fps[..., None], axis=1)

    px, py, pz = xyz_t[..., 0], xyz_t[..., 1], xyz_t[..., 2]
    norm2 = ((px * px + py * py) + pz * pz)[..., None]
    pad = jnp.zeros((B, N, _TABLE_D - 4 - points_t.shape[2]), f32)
    table = jnp.concatenate([xyz_t, norm2, points_t, pad], axis=2).reshape(B * N, _TABLE_D)
    boff = (jnp.arange(B, dtype=jnp.int32) * N)[:, None, None]
    sx, sy, sz = new_xyz[..., 0:1], new_xyz[..., 1:2], new_xyz[..., 2:3]
    sn = (sx * sx + sy * sy) + sz * sz

    outs = []
    for i, (radius, K) in enumerate(zip(_RADIUS, _NSAMPLE)):
        gi = _ball_query(xyz, new_xyz, radius, K)
        flat = (gi + boff).reshape(-1)
        rows = _gather_rows(table, flat).reshape(B, S, K, _TABLE_D)
        gx, gy, gz = rows[..., 0], rows[..., 1], rows[..., 2]
        mm = (sx * gx + sy * gy) + sz * gz
        sq = ((-2.0 * mm) + sn) + rows[..., 3]
        gxyz = rows[..., :3] - new_xyz[:, :, None, :]
        gpts = rows[..., 4:4 + points_t.shape[2]]
        yuan = jnp.broadcast_to(new_xyz[:, :, None, :], (B, S, K, 3))
        X = jnp.concatenate([gpts, gxyz, yuan, sq[..., None]], axis=-1)
        P = B * S * K
        Xf = jnp.transpose(X, (3, 0, 1, 2)).reshape(X.shape[-1], P)

        ac = None
        y = Xf
        for j, (W, b) in enumerate(params['convs'][i]):
            y, st = _layer(y, W, b[:, None], ac=ac)
            ac = _affine_from_stats(st, float(P))
        pooled = _maxpool(y, ac[0], ac[1], B * S, K)
        outs.append(jnp.transpose(pooled.reshape(-1, B, S), (1, 0, 2)))

    npc = jnp.concatenate(outs, axis=1)
    return (
        jnp.transpose(new_xyz, (0, 2, 1)),
        npc,
        jnp.transpose(new_color, (0, 2, 1)),
        npc,
    )
